# Initial kernel scaffold; baseline (speedup 1.0000x reference)
import jax
import jax.numpy as jnp
from jax.experimental import pallas as pl


def kernel(inputs, embedding, position_embedding):
    raise NotImplementedError("write your pallas kernel here")


# speedup vs baseline: 2.7427x; 2.7427x over previous
import functools

import jax
import jax.numpy as jnp
from jax import lax
from jax.experimental import pallas as pl
from jax.experimental.pallas import tpu as pltpu
from jax.experimental.pallas import tpu_sc as plsc

B = 4096
L = 200
H = 64
NC = 2
NS = 16
NW = NC * NS
SW = B // NW
IC0 = 128
IC1 = L - IC0


def _make_kernel():
    mesh = plsc.VectorSubcoreMesh(core_axis_name="c", subcore_axis_name="s")

    @functools.partial(
        pl.kernel,
        mesh=mesh,
        compiler_params=pltpu.CompilerParams(use_tc_tiling_on_sc=False),
        out_type=jax.ShapeDtypeStruct((B * L, 2 * H), jnp.float32),
        scratch_types=[
            pltpu.VMEM((L,), jnp.int32),
            pltpu.VMEM((L, H), jnp.float32),
            pltpu.VMEM((L, 2 * H), jnp.float32),
            pltpu.VMEM((L * H,), jnp.float32),
            pltpu.SemaphoreType.DMA,
        ],
    )
    def embed(idx_hbm, table_hbm, pos_hbm, out_hbm, idx_v, rows_v, out_v,
              pos_v, sem):
        wid = lax.axis_index("c") * NS + lax.axis_index("s")
        base = wid * SW

        pltpu.sync_copy(pos_hbm.at[pl.ds(0, L * H)], pos_v)

        def prefill(i, carry):
            for j in range(H // 16):
                out_v[i, pl.ds(j * 16, 16)] = pos_v[pl.ds(i * H + j * 16, 16)]
            return carry

        lax.fori_loop(0, L, prefill, 0)

        def body(s, carry):
            tok = (base + s) * L
            pltpu.sync_copy(idx_hbm.at[pl.ds(tok, L)], idx_v)
            pltpu.async_copy(
                table_hbm.at[idx_v.at[pl.ds(0, IC0)]],
                rows_v.at[pl.ds(0, IC0)],
                sem,
            ).wait()
            pltpu.async_copy(
                table_hbm.at[idx_v.at[pl.ds(IC0, IC1)]],
                rows_v.at[pl.ds(IC0, IC1)],
                sem,
            ).wait()

            def interleave(i, c):
                for j in range(H // 16):
                    out_v[i, pl.ds(H + j * 16, 16)] = rows_v[i, pl.ds(j * 16, 16)]
                return c

            lax.fori_loop(0, L, interleave, 0)
            pltpu.sync_copy(out_v, out_hbm.at[pl.ds(tok, L)])
            return carry

        lax.fori_loop(0, SW, body, 0)

    return embed


_embed = _make_kernel()


def kernel(inputs, embedding, position_embedding):
    idx = inputs.astype(jnp.int32).reshape(B * L)
    pos = position_embedding.reshape(-1)
    out = _embed(idx, embedding, pos)
    return out.reshape(B, L, 2 * H)


# speedup vs baseline: 3.9529x; 1.4412x over previous
import functools

import jax
import jax.numpy as jnp
from jax import lax
from jax.experimental import pallas as pl
from jax.experimental.pallas import tpu as pltpu
from jax.experimental.pallas import tpu_sc as plsc

B = 4096
L = 200
H = 64
NC = 2
NS = 16
NW = NC * NS
SW = B // NW
IC0 = 128
IC1 = L - IC0


def _make_kernel():
    mesh = plsc.VectorSubcoreMesh(core_axis_name="c", subcore_axis_name="s")

    @functools.partial(
        pl.kernel,
        mesh=mesh,
        compiler_params=pltpu.CompilerParams(use_tc_tiling_on_sc=False),
        out_type=jax.ShapeDtypeStruct((B * L, 2 * H), jnp.float32),
        scratch_types=[
            pltpu.VMEM((SW * L,), jnp.int32),
            pltpu.VMEM((2, L, H), jnp.float32),
            pltpu.VMEM((2, L, 2 * H), jnp.float32),
            pltpu.SemaphoreType.DMA,
            pltpu.SemaphoreType.DMA,
            pltpu.SemaphoreType.DMA,
            pltpu.SemaphoreType.DMA,
        ],
    )
    def embed(idx_hbm, table_hbm, pos_hbm, out_hbm, idx_v, rows_v, out_v,
              sem_g0, sem_g1, sem_w0, sem_w1):
        wid = lax.axis_index("c") * NS + lax.axis_index("s")
        base = wid * SW
        sem_g = (sem_g0, sem_g1)
        sem_w = (sem_w0, sem_w1)

        pltpu.sync_copy(idx_hbm.at[pl.ds(base * L, SW * L)], idx_v)

        pltpu.sync_copy(pos_hbm.at[pl.ds(0, L)], rows_v.at[0])

        def prefill(i, carry):
            for b in range(2):
                for j in range(H // 16):
                    out_v[b, i, pl.ds(j * 16, 16)] = rows_v[0, i, pl.ds(j * 16, 16)]
            return carry

        lax.fori_loop(0, L, prefill, 0)

        def gather_copies(s, b):
            off = s * L
            return (
                pltpu.make_async_copy(
                    table_hbm.at[idx_v.at[pl.ds(off, IC0)]],
                    rows_v.at[b].at[pl.ds(0, IC0)],
                    sem_g[b],
                ),
                pltpu.make_async_copy(
                    table_hbm.at[idx_v.at[pl.ds(off + IC0, IC1)]],
                    rows_v.at[b].at[pl.ds(IC0, IC1)],
                    sem_g[b],
                ),
            )

        def wb_copy(s, b):
            return pltpu.make_async_copy(
                out_v.at[b],
                out_hbm.at[pl.ds((base + s) * L, L)],
                sem_w[b],
            )

        def issue_gathers(s, b):
            for c in gather_copies(s, b):
                c.start()

        def interleave(b):
            def il(i, carry):
                for r in range(2):
                    row = i * 2 + r
                    for j in range(H // 16):
                        out_v[b, row, pl.ds(H + j * 16, 16)] = (
                            rows_v[b, row, pl.ds(j * 16, 16)]
                        )
                return carry

            lax.fori_loop(0, L // 2, il, 0)

        issue_gathers(0, 0)
        issue_gathers(1, 1)

        for b in range(2):
            for c in gather_copies(b, b):
                c.wait()
            interleave(b)
            wb_copy(b, b).start()
            issue_gathers(b + 2, b)

        def pair_body(g, carry):
            for b in range(2):
                s = 2 * g + b
                for c in gather_copies(s, b):
                    c.wait()
                wb_copy(s - 2, b).wait()
                interleave(b)
                wb_copy(s, b).start()

                @pl.when(s + 2 < SW)
                def _():
                    issue_gathers(s + 2, b)

            return carry

        lax.fori_loop(1, SW // 2, pair_body, 0)

        for b in range(2):
            wb_copy(SW - 2 + b, b).wait()

    return embed


_embed = _make_kernel()


def kernel(inputs, embedding, position_embedding):
    idx = inputs.astype(jnp.int32).reshape(B * L)
    out = _embed(idx, embedding, position_embedding)
    return out.reshape(B, L, 2 * H)
